# baseline (speedup 1.0000x reference)
import functools

import jax
import jax.numpy as jnp
from jax import lax
from jax.experimental import pallas as pl
from jax.experimental.pallas import tpu as pltpu

N = 100000
D = 128
BLK = 2048
GRID = (N + BLK - 1) // BLK

_KEY0 = 0
_KEY1 = 42
_KS2 = _KEY0 ^ _KEY1 ^ 0x1BD11BDA


def _rotl(x, d):
    return lax.shift_left(x, jnp.int32(d)) | lax.shift_right_logical(
        x, jnp.int32(32 - d)
    )


def _threefry2x32(x0, x1):
    ks = (jnp.int32(_KEY0), jnp.int32(_KEY1), jnp.int32(_KS2))
    rotations = ((13, 15, 26, 6), (17, 29, 16, 24))
    x0 = x0 + ks[0]
    x1 = x1 + ks[1]
    for i in range(5):
        for r in rotations[i % 2]:
            x0 = x0 + x1
            x1 = _rotl(x1, r)
            x1 = x0 ^ x1
        x0 = x0 + ks[(i + 1) % 3]
        x1 = x1 + ks[(i + 2) % 3] + jnp.int32(i + 1)
    return x0 ^ x1


def _body(x_ref, w_ref, b_ref, victim_ref, prob_ref, max_ref, idx_ref, sum_ref):
    k = pl.program_id(0)

    @pl.when(k == 0)
    def _init():
        max_ref[0] = jnp.int32(-1)
        idx_ref[0] = jnp.int32(N)
        sum_ref[0] = jnp.float32(0.0)

    rows = k * BLK + lax.broadcasted_iota(jnp.int32, (BLK, 1), 0)
    valid = rows < N

    s = jnp.sum(x_ref[...] * w_ref[...], axis=1, keepdims=True) + b_ref[0, 0]
    m = jnp.max(s, axis=1, keepdims=True)
    e = jnp.exp(s - m)
    p = e / jnp.sum(e, axis=1, keepdims=True)
    sum_ref[0] += jnp.sum(jnp.where(valid, p, 0.0))

    bits = _threefry2x32(jnp.zeros_like(rows), rows)
    key24 = jnp.where(valid, lax.shift_right_logical(bits, jnp.int32(9)),
                      jnp.int32(-1))
    bmax = jnp.max(key24)
    bidx = jnp.min(jnp.where(key24 == bmax, rows, jnp.int32(N)))

    better = bmax > max_ref[0]
    max_ref[0] = jnp.where(better, bmax, max_ref[0])
    idx_ref[0] = jnp.where(better, bidx, idx_ref[0])

    @pl.when(k == GRID - 1)
    def _fin():
        victim_ref[0] = idx_ref[0]
        prob_ref[0] = jnp.log(jnp.float32(1.0) / sum_ref[0])


@jax.jit
def _choose_victim(x, w, b):
    victim, prob = pl.pallas_call(
        _body,
        grid=(GRID,),
        in_specs=[
            pl.BlockSpec((BLK, D), lambda k: (k, 0)),
            pl.BlockSpec((1, D), lambda k: (0, 0)),
            pl.BlockSpec(memory_space=pltpu.SMEM),
        ],
        out_specs=[
            pl.BlockSpec(memory_space=pltpu.SMEM),
            pl.BlockSpec(memory_space=pltpu.SMEM),
        ],
        out_shape=[
            jax.ShapeDtypeStruct((1,), jnp.int32),
            jax.ShapeDtypeStruct((1,), jnp.float32),
        ],
        scratch_shapes=[
            pltpu.SMEM((1,), jnp.int32),
            pltpu.SMEM((1,), jnp.int32),
            pltpu.SMEM((1,), jnp.float32),
        ],
    )(x, w, b.reshape(1, 1))
    return victim[0], prob[0]


def kernel(x, W, b):
    return _choose_victim(x, W, b)


# speedup vs baseline: 88.9920x; 88.9920x over previous
import jax
import jax.numpy as jnp
from jax import lax
from jax.experimental import pallas as pl
from jax.experimental.pallas import tpu as pltpu

N = 100000
ROWS = (N + 127) // 128
ROWS_PAD = ((ROWS + 7) // 8) * 8

_KEY0 = 0
_KEY1 = 42
_KS2 = _KEY0 ^ _KEY1 ^ 0x1BD11BDA


def _rotl(x, d):
    return lax.shift_left(x, jnp.int32(d)) | lax.shift_right_logical(
        x, jnp.int32(32 - d)
    )


def _threefry2x32(x0, x1):
    ks = (jnp.int32(_KEY0), jnp.int32(_KEY1), jnp.int32(_KS2))
    rotations = ((13, 15, 26, 6), (17, 29, 16, 24))
    x0 = x0 + ks[0]
    x1 = x1 + ks[1]
    for i in range(5):
        for r in rotations[i % 2]:
            x0 = x0 + x1
            x1 = _rotl(x1, r)
            x1 = x0 ^ x1
        x0 = x0 + ks[(i + 1) % 3]
        x1 = x1 + ks[(i + 2) % 3] + jnp.int32(i + 1)
    return x0 ^ x1


def _body(victim_ref, prob_ref):
    ids = lax.broadcasted_iota(jnp.int32, (ROWS_PAD, 128), 0) * 128 \
        + lax.broadcasted_iota(jnp.int32, (ROWS_PAD, 128), 1)
    valid = ids < N

    bits = _threefry2x32(jnp.zeros_like(ids), ids)
    key24 = jnp.where(valid, lax.shift_right_logical(bits, jnp.int32(9)),
                      jnp.int32(-1))
    bmax = jnp.max(key24)
    victim_ref[0] = jnp.min(jnp.where(key24 == bmax, ids, jnp.int32(N)))

    sum_p = jnp.sum(jnp.where(valid, jnp.float32(1.0), jnp.float32(0.0)))
    prob_ref[0] = jnp.log(jnp.float32(1.0) / sum_p)


@jax.jit
def _choose_victim():
    victim, prob = pl.pallas_call(
        _body,
        out_specs=[
            pl.BlockSpec(memory_space=pltpu.SMEM),
            pl.BlockSpec(memory_space=pltpu.SMEM),
        ],
        out_shape=[
            jax.ShapeDtypeStruct((1,), jnp.int32),
            jax.ShapeDtypeStruct((1,), jnp.float32),
        ],
    )()
    return victim[0], prob[0]


def kernel(x, W, b):
    return _choose_victim()
